# baseline (speedup 1.0000x reference)
import jax
import jax.numpy as jnp
from jax import lax
from jax.experimental import pallas as pl
from jax.experimental.pallas import tpu as pltpu
from jax.experimental.pallas import tpu_sc as plsc

DM = 64
NC, NS = 2, 16
NW = NC * NS
CHUNK = 128
NBUF = 10

TB = 128
V = 1000000
NFULL = V // TB
TAILW = V - NFULL * TB
UNIT = (NFULL // NW) & ~1
NREM = NFULL - UNIT * NW


def _tr_block(tin_b, tout_b, rows, width):
    def vbody(v, carry):
        colv = jnp.full((16,), 0, jnp.int32) + v
        half = (v & 1) * DM
        for q in range(4):
            vals = plsc.load_gather(tin_b, [rows[q], colv])
            tout_b[v >> 1, pl.ds(half + q * 16, 16)] = vals
        return carry

    lax.fori_loop(0, width, vbody, 0)


def _relayout_body(tT_hbm, tail_hbm, scr_hbm, tin, tout, gsems, ssems):
    wid = lax.axis_index("s") * NC + lax.axis_index("c")
    iota = lax.iota(jnp.int32, 16)
    rows = [iota + q * 16 for q in range(4)]

    def g_off(g):
        return pl.multiple_of((wid + g * NW) * TB, TB)

    def start_in(g, b):
        return pltpu.async_copy(
            tT_hbm.at[:, pl.ds(g_off(g), TB)], tin.at[b], gsems.at[b])

    def start_out(g, b):
        row0 = pl.multiple_of((wid + g * NW) * (TB // 2), TB // 2)
        return pltpu.async_copy(
            tout.at[b], scr_hbm.at[pl.ds(row0, TB // 2)], ssems.at[b])

    def wait_store(b):
        pltpu.make_async_copy(
            tout.at[b], scr_hbm.at[pl.ds(0, TB // 2)], ssems.at[b]).wait()

    def wait_in(b):
        pltpu.make_async_copy(
            tT_hbm.at[:, pl.ds(0, TB)], tin.at[b], gsems.at[b]).wait()

    start_in(0, 0)
    start_in(1, 1)
    for b in (0, 1):
        wait_in(b)
        _tr_block(tin.at[b], tout.at[b], rows, TB)
        start_out(b, b)
        start_in(b + 2, b)

    def pair(ip, carry):
        for b in (0, 1):
            g = 2 * ip + b
            wait_in(b)
            wait_store(b)
            _tr_block(tin.at[b], tout.at[b], rows, TB)
            start_out(g, b)

            @pl.when(g + 2 < UNIT)
            def _():
                start_in(g + 2, b)

        return carry

    lax.fori_loop(1, UNIT // 2, pair, 0)
    wait_store(0)
    wait_store(1)

    @pl.when(wid < NREM)
    def _():
        v0 = pl.multiple_of((UNIT * NW + wid) * TB, TB)
        r0 = pl.multiple_of((UNIT * NW + wid) * (TB // 2), TB // 2)
        pltpu.sync_copy(tT_hbm.at[:, pl.ds(v0, TB)], tin.at[0])
        _tr_block(tin.at[0], tout.at[0], rows, TB)
        pltpu.sync_copy(tout.at[0], scr_hbm.at[pl.ds(r0, TB // 2)])

    @pl.when(wid == NREM)
    def _():
        tbuf = tout.at[0].at[pl.ds(0, TAILW // 2), :]
        pltpu.sync_copy(tail_hbm, tbuf)
        pltpu.sync_copy(tbuf, scr_hbm.at[pl.ds(NFULL * TB // 2, TAILW // 2)])


def _lookup_body(x_hbm, table_hbm, out_hbm, idx_v, rows_v, gsems, ssems):
    per_w = x_hbm.shape[0] // NW
    nch = per_w // CHUNK
    nrounds = nch // NBUF
    wid = lax.axis_index("s") * NC + lax.axis_index("c")
    base = pl.multiple_of(wid * per_w, per_w)
    pltpu.sync_copy(x_hbm.at[pl.ds(base, per_w)], idx_v)

    def gather(ci, b):
        off = ci * CHUNK
        return pltpu.async_copy(
            table_hbm.at[idx_v.at[pl.ds(off, CHUNK)]], rows_v.at[b],
            gsems.at[b])

    def store(ci, b):
        off = ci * CHUNK
        return pltpu.async_copy(
            rows_v.at[b], out_hbm.at[pl.ds(base + off, CHUNK)], ssems.at[b])

    def wait_store(b):
        pltpu.make_async_copy(
            rows_v.at[b], out_hbm.at[pl.ds(base, CHUNK)], ssems.at[b]).wait()

    g0 = [gather(b, b) for b in range(NBUF)]
    for b in range(NBUF):
        g0[b].wait()
        store(b, b)

    def round_body(r, carry):
        c0 = r * NBUF
        descs = []
        for b in range(NBUF):
            wait_store(b)
            descs.append(gather(c0 + b, b))
        for b in range(NBUF):
            descs[b].wait()
            store(c0 + b, b)
        return carry

    lax.fori_loop(1, nrounds, round_body, 0)

    for b in range(NBUF):
        wait_store(b)


def kernel(x, table):
    B, S = x.shape
    tot = B * S
    xf = x.reshape(tot).astype(jnp.int32)
    mesh = plsc.VectorSubcoreMesh(core_axis_name="c", subcore_axis_name="s")

    scr = pl.kernel(
        _relayout_body,
        out_type=jax.ShapeDtypeStruct((V // 2, 2 * DM), table.dtype),
        mesh=mesh,
        scratch_types=[
            pltpu.VMEM((2, DM, TB), jnp.float32),
            pltpu.VMEM((2, TB // 2, 2 * DM), jnp.float32),
            pltpu.SemaphoreType.DMA((2,)),
            pltpu.SemaphoreType.DMA((2,)),
        ],
        compiler_params=pltpu.CompilerParams(
            use_tc_tiling_on_sc=True, needs_layout_passes=False),
    )(table.T, table[NFULL * TB:, :].reshape(TAILW // 2, 2 * DM))

    tview = scr.reshape(V, DM)
    per_w = tot // NW
    out = pl.kernel(
        _lookup_body,
        out_type=jax.ShapeDtypeStruct((tot, DM), table.dtype),
        mesh=mesh,
        scratch_types=[
            pltpu.VMEM((per_w,), jnp.int32),
            pltpu.VMEM((NBUF, CHUNK, DM), jnp.float32),
            pltpu.SemaphoreType.DMA((NBUF,)),
            pltpu.SemaphoreType.DMA((NBUF,)),
        ],
        compiler_params=pltpu.CompilerParams(use_tc_tiling_on_sc=False),
    )(xf, tview)
    return out.reshape(B, S, DM)


# speedup vs baseline: 1.2024x; 1.2024x over previous
import jax
import jax.numpy as jnp
from jax import lax
from jax.experimental import pallas as pl
from jax.experimental.pallas import tpu as pltpu
from jax.experimental.pallas import tpu_sc as plsc

DM = 64
NC, NS = 2, 16
NW = NC * NS
CHUNK = 128
NBUF = 10

TB = 128
V = 1000000
NFULL = V // TB
TAILW = V - NFULL * TB
UNIT = (NFULL // NW) & ~1
NREM = NFULL - UNIT * NW


def _tr_block(tin_b, tout_b, scat_idx):
    def fbody(f, carry):
        for ci in range(TB // 16):
            vals = tin_b[f, pl.ds(ci * 16, 16)]
            rowv, colpat = scat_idx[ci]
            plsc.store_scatter(tout_b, [rowv, colpat + f], vals)
        return carry

    lax.fori_loop(0, DM, fbody, 0)


def _relayout_body(tT_hbm, tail_hbm, scr_hbm, tin, tout, gsems, ssems):
    wid = lax.axis_index("s") * NC + lax.axis_index("c")
    iota = lax.iota(jnp.int32, 16)
    scat_idx = []
    for ci in range(TB // 16):
        base = iota + ci * 16
        scat_idx.append((base >> 1, (base & 1) * DM))

    def g_off(g):
        return pl.multiple_of((wid + g * NW) * TB, TB)

    def start_in(g, b):
        return pltpu.async_copy(
            tT_hbm.at[:, pl.ds(g_off(g), TB)], tin.at[b], gsems.at[b])

    def start_out(g, b):
        row0 = pl.multiple_of((wid + g * NW) * (TB // 2), TB // 2)
        return pltpu.async_copy(
            tout.at[b], scr_hbm.at[pl.ds(row0, TB // 2)], ssems.at[b])

    def wait_store(b):
        pltpu.make_async_copy(
            tout.at[b], scr_hbm.at[pl.ds(0, TB // 2)], ssems.at[b]).wait()

    def wait_in(b):
        pltpu.make_async_copy(
            tT_hbm.at[:, pl.ds(0, TB)], tin.at[b], gsems.at[b]).wait()

    start_in(0, 0)
    start_in(1, 1)
    for b in (0, 1):
        wait_in(b)
        _tr_block(tin.at[b], tout.at[b], scat_idx)
        start_out(b, b)
        start_in(b + 2, b)

    def pair(ip, carry):
        for b in (0, 1):
            g = 2 * ip + b
            wait_in(b)
            wait_store(b)
            _tr_block(tin.at[b], tout.at[b], scat_idx)
            start_out(g, b)

            @pl.when(g + 2 < UNIT)
            def _():
                start_in(g + 2, b)

        return carry

    lax.fori_loop(1, UNIT // 2, pair, 0)
    wait_store(0)
    wait_store(1)

    @pl.when(wid < NREM)
    def _():
        v0 = pl.multiple_of((UNIT * NW + wid) * TB, TB)
        r0 = pl.multiple_of((UNIT * NW + wid) * (TB // 2), TB // 2)
        pltpu.sync_copy(tT_hbm.at[:, pl.ds(v0, TB)], tin.at[0])
        _tr_block(tin.at[0], tout.at[0], scat_idx)
        pltpu.sync_copy(tout.at[0], scr_hbm.at[pl.ds(r0, TB // 2)])

    @pl.when(wid == NREM)
    def _():
        tbuf = tout.at[0].at[pl.ds(0, TAILW // 2), :]
        pltpu.sync_copy(tail_hbm, tbuf)
        pltpu.sync_copy(tbuf, scr_hbm.at[pl.ds(NFULL * TB // 2, TAILW // 2)])


def _lookup_body(x_hbm, table_hbm, out_hbm, idx_v, rows_v, gsems, ssems):
    per_w = x_hbm.shape[0] // NW
    nch = per_w // CHUNK
    nrounds = nch // NBUF
    wid = lax.axis_index("s") * NC + lax.axis_index("c")
    base = pl.multiple_of(wid * per_w, per_w)
    pltpu.sync_copy(x_hbm.at[pl.ds(base, per_w)], idx_v)

    def gather(ci, b):
        off = ci * CHUNK
        return pltpu.async_copy(
            table_hbm.at[idx_v.at[pl.ds(off, CHUNK)]], rows_v.at[b],
            gsems.at[b])

    def store(ci, b):
        off = ci * CHUNK
        return pltpu.async_copy(
            rows_v.at[b], out_hbm.at[pl.ds(base + off, CHUNK)], ssems.at[b])

    def wait_store(b):
        pltpu.make_async_copy(
            rows_v.at[b], out_hbm.at[pl.ds(base, CHUNK)], ssems.at[b]).wait()

    g0 = [gather(b, b) for b in range(NBUF)]
    for b in range(NBUF):
        g0[b].wait()
        store(b, b)

    def round_body(r, carry):
        c0 = r * NBUF
        descs = []
        for b in range(NBUF):
            wait_store(b)
            descs.append(gather(c0 + b, b))
        for b in range(NBUF):
            descs[b].wait()
            store(c0 + b, b)
        return carry

    lax.fori_loop(1, nrounds, round_body, 0)

    for b in range(NBUF):
        wait_store(b)


def kernel(x, table):
    B, S = x.shape
    tot = B * S
    xf = x.reshape(tot).astype(jnp.int32)
    mesh = plsc.VectorSubcoreMesh(core_axis_name="c", subcore_axis_name="s")

    scr = pl.kernel(
        _relayout_body,
        out_type=jax.ShapeDtypeStruct((V // 2, 2 * DM), table.dtype),
        mesh=mesh,
        scratch_types=[
            pltpu.VMEM((2, DM, TB), jnp.float32),
            pltpu.VMEM((2, TB // 2, 2 * DM), jnp.float32),
            pltpu.SemaphoreType.DMA((2,)),
            pltpu.SemaphoreType.DMA((2,)),
        ],
        compiler_params=pltpu.CompilerParams(
            use_tc_tiling_on_sc=True, needs_layout_passes=False),
    )(table.T, table[NFULL * TB:, :].reshape(TAILW // 2, 2 * DM))

    tview = scr.reshape(V, DM)
    per_w = tot // NW
    out = pl.kernel(
        _lookup_body,
        out_type=jax.ShapeDtypeStruct((tot, DM), table.dtype),
        mesh=mesh,
        scratch_types=[
            pltpu.VMEM((per_w,), jnp.int32),
            pltpu.VMEM((NBUF, CHUNK, DM), jnp.float32),
            pltpu.SemaphoreType.DMA((NBUF,)),
            pltpu.SemaphoreType.DMA((NBUF,)),
        ],
        compiler_params=pltpu.CompilerParams(use_tc_tiling_on_sc=False),
    )(xf, tview)
    return out.reshape(B, S, DM)


# speedup vs baseline: 1.2084x; 1.0050x over previous
import jax
import jax.numpy as jnp
from jax import lax
from jax.experimental import pallas as pl
from jax.experimental.pallas import tpu as pltpu
from jax.experimental.pallas import tpu_sc as plsc

DM = 64
NC, NS = 2, 16
NW = NC * NS
CHUNK = 128
NBUF = 10
TCW = 512


def _tc_tr_body(tT_ref, out_ref):
    out_ref[:, 0:DM] = tT_ref[...].T


def _lookup_body(x_hbm, table_hbm, out_hbm, idx_v, rows_v, gsems, ssems):
    per_w = x_hbm.shape[0] // NW
    nch = per_w // CHUNK
    nrounds = nch // NBUF
    wid = lax.axis_index("s") * NC + lax.axis_index("c")
    base = pl.multiple_of(wid * per_w, per_w)
    pltpu.sync_copy(x_hbm.at[pl.ds(base, per_w)], idx_v)

    def dbl(j, carry):
        off = pl.multiple_of(j * 16, 16)
        idx_v[pl.ds(off, 16)] = idx_v[pl.ds(off, 16)] * 2
        return carry

    lax.fori_loop(0, per_w // 16, dbl, 0)

    def gather(ci, b):
        off = ci * CHUNK
        return pltpu.async_copy(
            table_hbm.at[idx_v.at[pl.ds(off, CHUNK)]], rows_v.at[b],
            gsems.at[b])

    def store(ci, b):
        off = ci * CHUNK
        return pltpu.async_copy(
            rows_v.at[b], out_hbm.at[pl.ds(base + off, CHUNK)], ssems.at[b])

    def wait_store(b):
        pltpu.make_async_copy(
            rows_v.at[b], out_hbm.at[pl.ds(base, CHUNK)], ssems.at[b]).wait()

    g0 = [gather(b, b) for b in range(NBUF)]
    for b in range(NBUF):
        g0[b].wait()
        store(b, b)

    def round_body(r, carry):
        c0 = r * NBUF
        descs = []
        for b in range(NBUF):
            wait_store(b)
            descs.append(gather(c0 + b, b))
        for b in range(NBUF):
            descs[b].wait()
            store(c0 + b, b)
        return carry

    lax.fori_loop(1, nrounds, round_body, 0)

    for b in range(NBUF):
        wait_store(b)


def kernel(x, table):
    B, S = x.shape
    tot = B * S
    V = table.shape[0]
    xf = x.reshape(tot).astype(jnp.int32)

    grid = (V + TCW - 1) // TCW
    tpad = pl.pallas_call(
        _tc_tr_body,
        grid=(grid,),
        in_specs=[pl.BlockSpec((DM, TCW), lambda i: (0, i))],
        out_specs=pl.BlockSpec((TCW, 2 * DM), lambda i: (i, 0)),
        out_shape=jax.ShapeDtypeStruct((V, 2 * DM), table.dtype),
    )(table.T)

    tview = tpad.reshape(2 * V, DM)
    per_w = tot // NW
    mesh = plsc.VectorSubcoreMesh(core_axis_name="c", subcore_axis_name="s")
    out = pl.kernel(
        _lookup_body,
        out_type=jax.ShapeDtypeStruct((tot, DM), table.dtype),
        mesh=mesh,
        scratch_types=[
            pltpu.VMEM((per_w,), jnp.int32),
            pltpu.VMEM((NBUF, CHUNK, DM), jnp.float32),
            pltpu.SemaphoreType.DMA((NBUF,)),
            pltpu.SemaphoreType.DMA((NBUF,)),
        ],
        compiler_params=pltpu.CompilerParams(use_tc_tiling_on_sc=False),
    )(xf, tview)
    return out.reshape(B, S, DM)


# speedup vs baseline: 2.3894x; 1.9773x over previous
import jax
import jax.numpy as jnp
from jax import lax
from jax.experimental import pallas as pl
from jax.experimental.pallas import tpu as pltpu
from jax.experimental.pallas import tpu_sc as plsc

DM = 64
NC, NS = 2, 16
NW = NC * NS
CHUNK = 128
NBUF = 10
TCW = 2048


def _tc_tr_body(tT_ref, out_ref):
    t = tT_ref[...].T
    out_ref[...] = jnp.concatenate([t, t], axis=1)


def _lookup_body(x_hbm, table_hbm, out_hbm, idx_v, rows_v, gsems, ssems):
    per_w = x_hbm.shape[0] // NW
    nch = per_w // CHUNK
    nrounds = nch // NBUF
    wid = lax.axis_index("s") * NC + lax.axis_index("c")
    base = pl.multiple_of(wid * per_w, per_w)
    pltpu.sync_copy(x_hbm.at[pl.ds(base, per_w)], idx_v)

    def dbl(j, carry):
        off = pl.multiple_of(j * 16, 16)
        idx_v[pl.ds(off, 16)] = idx_v[pl.ds(off, 16)] * 2
        return carry

    lax.fori_loop(0, per_w // 16, dbl, 0)

    def gather(ci, b):
        off = ci * CHUNK
        return pltpu.async_copy(
            table_hbm.at[idx_v.at[pl.ds(off, CHUNK)]], rows_v.at[b],
            gsems.at[b])

    def store(ci, b):
        off = ci * CHUNK
        return pltpu.async_copy(
            rows_v.at[b], out_hbm.at[pl.ds(base + off, CHUNK)], ssems.at[b])

    def wait_store(b):
        pltpu.make_async_copy(
            rows_v.at[b], out_hbm.at[pl.ds(base, CHUNK)], ssems.at[b]).wait()

    g0 = [gather(b, b) for b in range(NBUF)]
    for b in range(NBUF):
        g0[b].wait()
        store(b, b)

    def round_body(r, carry):
        c0 = r * NBUF
        descs = []
        for b in range(NBUF):
            wait_store(b)
            descs.append(gather(c0 + b, b))
        for b in range(NBUF):
            descs[b].wait()
            store(c0 + b, b)
        return carry

    lax.fori_loop(1, nrounds, round_body, 0)

    for b in range(NBUF):
        wait_store(b)


def kernel(x, table):
    B, S = x.shape
    tot = B * S
    V = table.shape[0]
    xf = x.reshape(tot).astype(jnp.int32)

    grid = (V + TCW - 1) // TCW
    tpad = pl.pallas_call(
        _tc_tr_body,
        grid=(grid,),
        in_specs=[pl.BlockSpec((DM, TCW), lambda i: (0, i))],
        out_specs=pl.BlockSpec((TCW, 2 * DM), lambda i: (i, 0)),
        out_shape=jax.ShapeDtypeStruct((V, 2 * DM), table.dtype),
    )(table.T)

    tview = tpad.reshape(2 * V, DM)
    per_w = tot // NW
    mesh = plsc.VectorSubcoreMesh(core_axis_name="c", subcore_axis_name="s")
    out = pl.kernel(
        _lookup_body,
        out_type=jax.ShapeDtypeStruct((tot, DM), table.dtype),
        mesh=mesh,
        scratch_types=[
            pltpu.VMEM((per_w,), jnp.int32),
            pltpu.VMEM((NBUF, CHUNK, DM), jnp.float32),
            pltpu.SemaphoreType.DMA((NBUF,)),
            pltpu.SemaphoreType.DMA((NBUF,)),
        ],
        compiler_params=pltpu.CompilerParams(use_tc_tiling_on_sc=False),
    )(xf, tview)
    return out.reshape(B, S, DM)


# speedup vs baseline: 2.5837x; 1.0813x over previous
import jax
import jax.numpy as jnp
from jax import lax
from jax.experimental import pallas as pl
from jax.experimental.pallas import tpu as pltpu
from jax.experimental.pallas import tpu_sc as plsc

DM = 64
NC, NS = 2, 16
NW = NC * NS
CHUNK = 128
NBUF = 10
TCW = 2048


def _tc_tr_body(tT_ref, out_ref):
    out_ref[:, 0:DM] = tT_ref[...].T


def _lookup_body(x_hbm, table_hbm, out_hbm, idx_v, rows_v, gsems, ssems):
    per_w = x_hbm.shape[0] // NW
    nch = per_w // CHUNK
    nrounds = nch // NBUF
    wid = lax.axis_index("s") * NC + lax.axis_index("c")
    base = pl.multiple_of(wid * per_w, per_w)
    pltpu.sync_copy(x_hbm.at[pl.ds(base, per_w)], idx_v)

    def dbl(j, carry):
        off = pl.multiple_of(j * 16, 16)
        idx_v[pl.ds(off, 16)] = idx_v[pl.ds(off, 16)] * 2
        return carry

    lax.fori_loop(0, per_w // 16, dbl, 0)

    def gather(ci, b):
        off = ci * CHUNK
        return pltpu.async_copy(
            table_hbm.at[idx_v.at[pl.ds(off, CHUNK)]], rows_v.at[b],
            gsems.at[b])

    def store(ci, b):
        off = ci * CHUNK
        return pltpu.async_copy(
            rows_v.at[b], out_hbm.at[pl.ds(base + off, CHUNK)], ssems.at[b])

    def wait_store(b):
        pltpu.make_async_copy(
            rows_v.at[b], out_hbm.at[pl.ds(base, CHUNK)], ssems.at[b]).wait()

    g0 = [gather(b, b) for b in range(NBUF)]
    for b in range(NBUF):
        g0[b].wait()
        store(b, b)

    def round_body(r, carry):
        c0 = r * NBUF
        descs = []
        for b in range(NBUF):
            wait_store(b)
            descs.append(gather(c0 + b, b))
        for b in range(NBUF):
            descs[b].wait()
            store(c0 + b, b)
        return carry

    lax.fori_loop(1, nrounds, round_body, 0)

    for b in range(NBUF):
        wait_store(b)


def kernel(x, table):
    B, S = x.shape
    tot = B * S
    V = table.shape[0]
    xf = x.reshape(tot).astype(jnp.int32)

    grid = (V + TCW - 1) // TCW
    tpad = pl.pallas_call(
        _tc_tr_body,
        grid=(grid,),
        in_specs=[pl.BlockSpec((DM, TCW), lambda i: (0, i))],
        out_specs=pl.BlockSpec((TCW, 2 * DM), lambda i: (i, 0)),
        out_shape=jax.ShapeDtypeStruct((V, 2 * DM), table.dtype),
    )(table.T)

    tview = tpad.reshape(2 * V, DM)
    per_w = tot // NW
    mesh = plsc.VectorSubcoreMesh(core_axis_name="c", subcore_axis_name="s")
    out = pl.kernel(
        _lookup_body,
        out_type=jax.ShapeDtypeStruct((tot, DM), table.dtype),
        mesh=mesh,
        scratch_types=[
            pltpu.VMEM((per_w,), jnp.int32),
            pltpu.VMEM((NBUF, CHUNK, DM), jnp.float32),
            pltpu.SemaphoreType.DMA((NBUF,)),
            pltpu.SemaphoreType.DMA((NBUF,)),
        ],
        compiler_params=pltpu.CompilerParams(use_tc_tiling_on_sc=False),
    )(xf, tview)
    return out.reshape(B, S, DM)


# speedup vs baseline: 3.7093x; 1.4357x over previous
import jax
import jax.numpy as jnp
from jax import lax
from jax.experimental import pallas as pl
from jax.experimental.pallas import tpu as pltpu
from jax.experimental.pallas import tpu_sc as plsc

DM = 64
NC, NS = 2, 16
NW = NC * NS
CHUNK = 128
NBUF = 10
TCW = 8192


def _tc_tr_body(tT_ref, out_ref):
    out_ref[:, 0:DM] = tT_ref[...].T


def _lookup_body(x_hbm, table_hbm, out_hbm, idx_v, rows_v, gsems, ssems):
    per_w = x_hbm.shape[0] // NW
    nch = per_w // CHUNK
    nrounds = nch // NBUF
    wid = lax.axis_index("s") * NC + lax.axis_index("c")
    base = pl.multiple_of(wid * per_w, per_w)
    pltpu.sync_copy(x_hbm.at[pl.ds(base, per_w)], idx_v)

    def dbl(j, carry):
        off = pl.multiple_of(j * 16, 16)
        idx_v[pl.ds(off, 16)] = idx_v[pl.ds(off, 16)] * 2
        return carry

    lax.fori_loop(0, per_w // 16, dbl, 0)

    def gather(ci, b):
        off = ci * CHUNK
        return pltpu.async_copy(
            table_hbm.at[idx_v.at[pl.ds(off, CHUNK)]], rows_v.at[b],
            gsems.at[b])

    def store(ci, b):
        off = ci * CHUNK
        return pltpu.async_copy(
            rows_v.at[b], out_hbm.at[pl.ds(base + off, CHUNK)], ssems.at[b])

    def wait_store(b):
        pltpu.make_async_copy(
            rows_v.at[b], out_hbm.at[pl.ds(base, CHUNK)], ssems.at[b]).wait()

    g0 = [gather(b, b) for b in range(NBUF)]
    for b in range(NBUF):
        g0[b].wait()
        store(b, b)

    def round_body(r, carry):
        c0 = r * NBUF
        descs = []
        for b in range(NBUF):
            wait_store(b)
            descs.append(gather(c0 + b, b))
        for b in range(NBUF):
            descs[b].wait()
            store(c0 + b, b)
        return carry

    lax.fori_loop(1, nrounds, round_body, 0)

    for b in range(NBUF):
        wait_store(b)


def kernel(x, table):
    B, S = x.shape
    tot = B * S
    V = table.shape[0]
    xf = x.reshape(tot).astype(jnp.int32)

    grid = (V + TCW - 1) // TCW
    tpad = pl.pallas_call(
        _tc_tr_body,
        grid=(grid,),
        in_specs=[pl.BlockSpec((DM, TCW), lambda i: (0, i))],
        out_specs=pl.BlockSpec((TCW, 2 * DM), lambda i: (i, 0)),
        out_shape=jax.ShapeDtypeStruct((V, 2 * DM), table.dtype),
    )(table.T)

    tview = tpad.reshape(2 * V, DM)
    per_w = tot // NW
    mesh = plsc.VectorSubcoreMesh(core_axis_name="c", subcore_axis_name="s")
    out = pl.kernel(
        _lookup_body,
        out_type=jax.ShapeDtypeStruct((tot, DM), table.dtype),
        mesh=mesh,
        scratch_types=[
            pltpu.VMEM((per_w,), jnp.int32),
            pltpu.VMEM((NBUF, CHUNK, DM), jnp.float32),
            pltpu.SemaphoreType.DMA((NBUF,)),
            pltpu.SemaphoreType.DMA((NBUF,)),
        ],
        compiler_params=pltpu.CompilerParams(use_tc_tiling_on_sc=False),
    )(xf, tview)
    return out.reshape(B, S, DM)
